# baseline (speedup 1.0000x reference)
import functools

import jax
import jax.numpy as jnp
from jax import lax
from jax.experimental import pallas as pl
from jax.experimental.pallas import tpu as pltpu
from jax.experimental.pallas import tpu_sc as plsc

F32 = jnp.float32
I32 = jnp.int32

N_NODES = 10000
N_PAD = 10240
N_EDGES = 320000
N_TRI = 320000
H = 128
BLK = 512
E_GRID = N_EDGES // BLK
N_GRID = N_PAD // BLK
CHUNK = 128
DR = 6400
NWIN = N_EDGES // DR
NROUND = 25
DRN = 5120
NWINN = N_PAD // DRN


def _swish(v):
    return v * jax.nn.sigmoid(v)


def _rows(cols):
    return pl.BlockSpec((BLK, cols), lambda ii: (ii, 0))


def _full(shape):
    nd = len(shape)
    return pl.BlockSpec(shape, lambda ii: (0,) * nd)


def _nodeemb_body(z_ref, emb_ref, out_ref):
    zb = z_ref[0, 0]
    oh = (zb[:, None] == lax.broadcasted_iota(I32, (1, 95), 1)).astype(F32)
    out_ref[...] = jnp.dot(oh, emb_ref[...], preferred_element_type=F32)


def _node_embed(z3, emb):
    return pl.pallas_call(
        _nodeemb_body,
        grid=(N_GRID,),
        in_specs=[pl.BlockSpec((1, 1, BLK), lambda ii: (ii, 0, 0)), _full((95, H))],
        out_specs=_rows(H),
        out_shape=jax.ShapeDtypeStruct((N_PAD, H), F32),
    )(z3, emb)


def _embed_body(xi, xj, rbf, Wr, br, W1, W2, W3, be, orbf, ox, ot):
    rb = rbf[...]
    re = _swish(jnp.dot(rb, Wr[...], preferred_element_type=F32) + br[...])
    acc = (jnp.dot(xi[...], W1[...], preferred_element_type=F32)
           + jnp.dot(xj[...], W2[...], preferred_element_type=F32)
           + jnp.dot(re, W3[...], preferred_element_type=F32) + be[...])
    xv = _swish(acc)
    ox[...] = xv
    ot[...] = jnp.dot(rb, orbf[...], preferred_element_type=F32) * xv


def _embed(xi, xj, rbf, Wr, br, W1, W2, W3, be, orbf):
    return pl.pallas_call(
        _embed_body,
        grid=(E_GRID,),
        in_specs=[_rows(H), _rows(H), _rows(6), _full((6, H)), _full((1, H)),
                  _full((H, H)), _full((H, H)), _full((H, H)), _full((1, H)),
                  _full((6, H))],
        out_specs=[_rows(H), _rows(H)],
        out_shape=[jax.ShapeDtypeStruct((N_EDGES, H), F32),
                   jax.ShapeDtypeStruct((N_EDGES, H), F32)],
    )(xi, xj, rbf, Wr, br, W1, W2, W3, be, orbf)


def _pre_body(x, rbf, Wji, bji, Wkj, bkj, lr, oji, okj):
    xb = x[...]
    oji[...] = _swish(jnp.dot(xb, Wji[...], preferred_element_type=F32) + bji[...])
    okj[...] = (_swish(jnp.dot(xb, Wkj[...], preferred_element_type=F32) + bkj[...])
                * jnp.dot(rbf[...], lr[...], preferred_element_type=F32))


def _pre(x, rbf, Wji, bji, Wkj, bkj, lr):
    return pl.pallas_call(
        _pre_body,
        grid=(E_GRID,),
        in_specs=[_rows(H), _rows(6), _full((H, H)), _full((1, H)),
                  _full((H, H)), _full((1, H)), _full((6, H))],
        out_specs=[_rows(H), _rows(H)],
        out_shape=[jax.ShapeDtypeStruct((N_EDGES, H), F32),
                   jax.ShapeDtypeStruct((N_EDGES, H), F32)],
    )(x, rbf, Wji, bji, Wkj, bkj, lr)


def _tri_body(g, sbf, ls, U8, out):
    sb = jnp.dot(sbf[...], ls[...], preferred_element_type=F32)
    gb = g[...]
    acc = jnp.zeros((BLK, H), F32)
    for jj in range(8):
        acc = acc + sb[:, jj:jj + 1] * jnp.dot(gb, U8[jj], preferred_element_type=F32)
    out[...] = acc


def _tri(g, sbf, ls, U8):
    return pl.pallas_call(
        _tri_body,
        grid=(E_GRID,),
        in_specs=[_rows(H), _rows(42), _full((42, 8)), _full((8, H, H))],
        out_specs=_rows(H),
        out_shape=jax.ShapeDtypeStruct((N_TRI, H), F32),
    )(g, sbf, ls, U8)


def _post_body(agg, xji, x, rbf, rb0W, rb0b, rb1W, rb1b, WlI, blI,
               q0W, q0b, q1W, q1b, q2W, q2b, q3W, q3b, orbf, ox, ot):
    h = xji[...] + agg[...]
    h1 = _swish(jnp.dot(h, rb0W[...], preferred_element_type=F32) + rb0b[...])
    h = h + _swish(jnp.dot(h1, rb1W[...], preferred_element_type=F32) + rb1b[...])
    h = _swish(jnp.dot(h, WlI[...], preferred_element_type=F32) + blI[...]) + x[...]
    h1 = _swish(jnp.dot(h, q0W[...], preferred_element_type=F32) + q0b[...])
    h = h + _swish(jnp.dot(h1, q1W[...], preferred_element_type=F32) + q1b[...])
    h1 = _swish(jnp.dot(h, q2W[...], preferred_element_type=F32) + q2b[...])
    h = h + _swish(jnp.dot(h1, q3W[...], preferred_element_type=F32) + q3b[...])
    ox[...] = h
    ot[...] = jnp.dot(rbf[...], orbf[...], preferred_element_type=F32) * h


def _post(agg, xji, x, rbf, weights):
    wspecs = [_full((H, H)), _full((1, H))] * 7
    return pl.pallas_call(
        _post_body,
        grid=(E_GRID,),
        in_specs=[_rows(H), _rows(H), _rows(H), _rows(6)] + wspecs + [_full((6, H))],
        out_specs=[_rows(H), _rows(H)],
        out_shape=[jax.ShapeDtypeStruct((N_EDGES, H), F32),
                   jax.ShapeDtypeStruct((N_EDGES, H), F32)],
    )(agg, xji, x, rbf, *weights)


def _nmlp_body(ns, W0, b0, W1, b1, W2, b2, olp, out):
    n = ns[...]
    n = _swish(jnp.dot(n, W0[...], preferred_element_type=F32) + b0[...])
    n = _swish(jnp.dot(n, W1[...], preferred_element_type=F32) + b1[...])
    n = _swish(jnp.dot(n, W2[...], preferred_element_type=F32) + b2[...])
    out[...] = jnp.dot(n, olp[...], preferred_element_type=F32)


def _nmlp(ns, W0, b0, W1, b1, W2, b2, olp):
    return pl.pallas_call(
        _nmlp_body,
        grid=(N_GRID,),
        in_specs=[_rows(H),
                  _full((H, H)), _full((1, H)), _full((H, H)), _full((1, H)),
                  _full((H, H)), _full((1, H)), _full((H, H))],
        out_specs=_rows(H),
        out_shape=jax.ShapeDtypeStruct((N_PAD, H), F32),
    )(ns, W0, b0, W1, b1, W2, b2, olp)


def _mesh():
    return plsc.VectorSubcoreMesh(core_axis_name="c", subcore_axis_name="s")


GRP = 4 * CHUNK


def _sc_gather(table, idx4):
    ngrp = idx4.shape[0]

    @functools.partial(
        pl.kernel,
        out_type=jax.ShapeDtypeStruct((ngrp * GRP, H), F32),
        mesh=_mesh(),
        compiler_params=pltpu.CompilerParams(needs_layout_passes=False),
        scratch_types=[pltpu.VMEM((4, CHUNK), I32),
                       pltpu.VMEM((GRP, H), F32),
                       pltpu.SemaphoreType.DMA],
    )
    def k(tbl, idx, out, iv, buf, sem):
        wid = lax.axis_index("s") * 2 + lax.axis_index("c")
        nk = (ngrp - wid + 31) // 32

        def body(it, carry):
            grp = wid + it * 32
            pltpu.sync_copy(idx.at[grp], iv)
            cps = [pltpu.async_copy(tbl.at[iv.at[b]],
                                    buf.at[pl.ds(b * CHUNK, CHUNK)], sem)
                   for b in range(4)]
            for cp in cps:
                cp.wait()
            pltpu.sync_copy(buf, out.at[pl.ds(grp * GRP, GRP)])
            return carry

        lax.fori_loop(0, nk, body, 0)

    return k(table, idx4)


def _sc_scatter_sorted(m, pexp, ldexp, blo, bhi, zrows, dr, nround, nout):
    ngexp = pexp.shape[0]
    step = dr // 16
    nwin = nout // dr

    @functools.partial(
        pl.kernel,
        out_type=jax.ShapeDtypeStruct((nout, H), F32),
        mesh=_mesh(),
        compiler_params=pltpu.CompilerParams(needs_layout_passes=False),
        scratch_types=[pltpu.VMEM((4, CHUNK), I32),
                       pltpu.VMEM((4, CHUNK), I32),
                       pltpu.VMEM((GRP, H), F32),
                       pltpu.VMEM((16,), I32),
                       pltpu.VMEM((16,), I32),
                       pltpu.VMEM((16,), I32),
                       pltpu.VMEM((16,), I32),
                       pltpu.VMEM_SHARED((dr + 8, H), F32),
                       pltpu.SemaphoreType.DMA],
    )
    def k(mref, pref, ldref, blo_r, bhi_r, zr, out,
          iv, ldv, buf, blv0, blv1, bhv0, bhv1, spm, sem):
        c = lax.axis_index("c")
        sid = lax.axis_index("s")
        pltpu.sync_copy(blo_r.at[c, 0], blv0)
        pltpu.sync_copy(blo_r.at[c, 1], blv1)
        pltpu.sync_copy(bhi_r.at[c, 0], bhv0)
        pltpu.sync_copy(bhi_r.at[c, 1], bhv1)
        lane = lax.iota(I32, 16)
        for r in range(nround):
            g = c * nround + r
            base = g * dr
            blvec = (blv0 if r < 16 else blv1)[...]
            bhvec = (bhv0 if r < 16 else bhv1)[...]
            sel = lane == (r % 16)
            c0 = jnp.clip(jnp.max(jnp.where(sel, blvec, 0)), 0, ngexp)
            c1 = jnp.clip(jnp.max(jnp.where(sel, bhvec, 0)), 0, ngexp)

            @pl.when(g < nwin)
            def _round():
                pltpu.sync_copy(zr.at[pl.ds(0, step)],
                                spm.at[pl.ds(sid * step, step)])
                plsc.subcore_barrier()
                nk = jnp.maximum(0, (c1 - c0 - sid + 15) // 16)

                def body(it, carry):
                    grp = jnp.clip(c0 + sid + it * 16, 0, ngexp - 1)
                    pltpu.sync_copy(pref.at[grp], iv)
                    pltpu.sync_copy(ldref.at[grp], ldv)
                    cps = [pltpu.async_copy(mref.at[iv.at[b]],
                                            buf.at[pl.ds(b * CHUNK, CHUNK)], sem)
                           for b in range(4)]
                    for cp in cps:
                        cp.wait()
                    cps = [pltpu.async_copy(buf.at[pl.ds(b * CHUNK, CHUNK)],
                                            spm.at[ldv.at[b]], sem, add=True)
                           for b in range(4)]
                    for cp in cps:
                        cp.wait()
                    return carry

                lax.fori_loop(0, nk, body, 0)
                plsc.subcore_barrier()
                pltpu.sync_copy(spm.at[pl.ds(sid * step, step)],
                                out.at[pl.ds(base + sid * step, step)])

    return k(m, pexp, ldexp, blo, bhi, zrows)


def _scatter_plan(dest, dr, nwin, nround):
    n = dest.shape[0]
    perm = jnp.argsort(dest).astype(I32)
    s = jnp.take(dest, perm)
    bounds = jnp.searchsorted(s, jnp.arange(nwin + 1) * dr).astype(I32)
    nwin_sz = bounds[1:] - bounds[:-1]
    npad = ((nwin_sz + GRP - 1) // GRP) * GRP
    start_p = jnp.concatenate([jnp.zeros((1,), I32), jnp.cumsum(npad).astype(I32)])
    w_of = s // dr
    pos = jnp.arange(n, dtype=I32) - bounds[w_of] + start_p[w_of]
    ngexp = n // GRP + nwin
    texp = ngexp * GRP
    pexp = jnp.zeros((texp,), I32).at[pos].set(perm).reshape(ngexp, 4, CHUNK)
    ldexp = jnp.full((texp,), dr, I32).at[pos].set(s % dr).reshape(ngexp, 4, CHUNK)
    cb = start_p // GRP

    def pack(a, b):
        return jnp.stack([jnp.pad(a, (0, 32 - a.shape[0])),
                          jnp.pad(b, (0, 32 - b.shape[0]))]).reshape(2, 2, 16)

    blo = pack(cb[0:nround], cb[nround:nwin])
    bhi = pack(cb[1:nround + 1], cb[nround + 1:nwin + 1])
    return pexp, ldexp, blo, bhi


def kernel(z, rbf, sbf, i, j, idx_kj, idx_ji, params):
    p = params
    i = i.astype(I32)
    j = j.astype(I32)
    idx_kj = idx_kj.astype(I32)
    idx_ji = idx_ji.astype(I32)

    z3 = jnp.pad(z.astype(I32), (0, N_PAD - N_NODES)).reshape(N_GRID, 1, BLK)
    ij2 = jnp.concatenate([i, j]).reshape((2 * N_EDGES) // GRP, 4, CHUNK)
    kj2 = idx_kj.reshape(N_TRI // GRP, 4, CHUNK)
    peE, ldE, bloE, bhiE = _scatter_plan(idx_ji, DR, NWIN, NROUND)
    peN, ldN, bloN, bhiN = _scatter_plan(i, DRN, NWINN, 1)
    zrows = jnp.zeros((DR // 16, H), F32)

    def b1(v):
        return v.reshape(1, H)

    W1 = p['emb_lin_W'][0:H]
    W2 = p['emb_lin_W'][H:2 * H]
    W3 = p['emb_lin_W'][2 * H:3 * H]

    xn = _node_embed(z3, p['emb'])
    xij = _sc_gather(xn, ij2)
    xi = xij[:N_EDGES]
    xj = xij[N_EDGES:]
    x, t = _embed(xi, xj, rbf, p['emb_lin_rbf_W'], b1(p['emb_lin_rbf_b']),
                  W1, W2, W3, b1(p['emb_lin_b']), p['out_lin_rbf'][0])

    def out_block(bb, t_e):
        ns = _sc_scatter_sorted(t_e, peN, ldN, bloN, bhiN, zrows, DRN, 1, N_PAD)
        olp = jnp.pad(p['out_lin'][bb], ((0, 0), (0, H - p['out_lin'][bb].shape[1])))
        return _nmlp(ns, p['out_lins_W'][bb, 0], b1(p['out_lins_b'][bb, 0]),
                     p['out_lins_W'][bb, 1], b1(p['out_lins_b'][bb, 1]),
                     p['out_lins_W'][bb, 2], b1(p['out_lins_b'][bb, 2]), olp)

    P = out_block(0, t)
    for b in range(6):
        xji, xkj = _pre(x, rbf, p['int_lin_ji_W'][b], b1(p['int_lin_ji_b'][b]),
                        p['int_lin_kj_W'][b], b1(p['int_lin_kj_b'][b]),
                        p['int_lin_rbf'][b])
        g = _sc_gather(xkj, kj2)
        U8 = jnp.transpose(p['int_W'][b], (1, 2, 0))
        m = _tri(g, sbf, p['int_lin_sbf'][b], U8)
        agg = _sc_scatter_sorted(m, peE, ldE, bloE, bhiE, zrows, DR, NROUND, N_EDGES)
        weights = [
            p['res_before_W'][b, 0, 0], b1(p['res_before_b'][b, 0, 0]),
            p['res_before_W'][b, 0, 1], b1(p['res_before_b'][b, 0, 1]),
            p['int_lin_W'][b], b1(p['int_lin_b'][b]),
            p['res_after_W'][b, 0, 0], b1(p['res_after_b'][b, 0, 0]),
            p['res_after_W'][b, 0, 1], b1(p['res_after_b'][b, 0, 1]),
            p['res_after_W'][b, 1, 0], b1(p['res_after_b'][b, 1, 0]),
            p['res_after_W'][b, 1, 1], b1(p['res_after_b'][b, 1, 1]),
            p['out_lin_rbf'][b + 1],
        ]
        x, t = _post(agg, xji, x, rbf, weights)
        P = P + out_block(b + 1, t)
    return P[:N_NODES, 0:1]


# speedup vs baseline: 1.0544x; 1.0544x over previous
import functools

import jax
import jax.numpy as jnp
from jax import lax
from jax.experimental import pallas as pl
from jax.experimental.pallas import tpu as pltpu
from jax.experimental.pallas import tpu_sc as plsc

F32 = jnp.float32
I32 = jnp.int32

N_NODES = 10000
N_PAD = 10240
N_EDGES = 320000
N_TRI = 320000
H = 128
BLK = 512
E_GRID = N_EDGES // BLK
N_GRID = N_PAD // BLK
CHUNK = 128
DR = 6400
NWIN = N_EDGES // DR
NROUND = 25
DRN = 5120
NWINN = N_PAD // DRN


def _swish(v):
    return v * jax.nn.sigmoid(v)


def _rows(cols):
    return pl.BlockSpec((BLK, cols), lambda ii: (ii, 0))


def _full(shape):
    nd = len(shape)
    return pl.BlockSpec(shape, lambda ii: (0,) * nd)


def _nodeemb_body(z_ref, emb_ref, out_ref):
    zb = z_ref[0, 0]
    oh = (zb[:, None] == lax.broadcasted_iota(I32, (1, 95), 1)).astype(F32)
    out_ref[...] = jnp.dot(oh, emb_ref[...], preferred_element_type=F32)


def _node_embed(z3, emb):
    return pl.pallas_call(
        _nodeemb_body,
        grid=(N_GRID,),
        in_specs=[pl.BlockSpec((1, 1, BLK), lambda ii: (ii, 0, 0)), _full((95, H))],
        out_specs=_rows(H),
        out_shape=jax.ShapeDtypeStruct((N_PAD, H), F32),
    )(z3, emb)


def _embed_body(xi, xj, rbf, Wr, br, W1, W2, W3, be, orbf, ox, ot):
    rb = rbf[...]
    re = _swish(jnp.dot(rb, Wr[...], preferred_element_type=F32) + br[...])
    acc = (jnp.dot(xi[...], W1[...], preferred_element_type=F32)
           + jnp.dot(xj[...], W2[...], preferred_element_type=F32)
           + jnp.dot(re, W3[...], preferred_element_type=F32) + be[...])
    xv = _swish(acc)
    ox[...] = xv
    ot[...] = jnp.dot(rb, orbf[...], preferred_element_type=F32) * xv


def _embed(xi, xj, rbf, Wr, br, W1, W2, W3, be, orbf):
    return pl.pallas_call(
        _embed_body,
        grid=(E_GRID,),
        in_specs=[_rows(H), _rows(H), _rows(6), _full((6, H)), _full((1, H)),
                  _full((H, H)), _full((H, H)), _full((H, H)), _full((1, H)),
                  _full((6, H))],
        out_specs=[_rows(H), _rows(H)],
        out_shape=[jax.ShapeDtypeStruct((N_EDGES, H), F32),
                   jax.ShapeDtypeStruct((N_EDGES, H), F32)],
    )(xi, xj, rbf, Wr, br, W1, W2, W3, be, orbf)


def _pre_body(x, rbf, Wji, bji, Wkj, bkj, lr, oji, okj):
    xb = x[...]
    oji[...] = _swish(jnp.dot(xb, Wji[...], preferred_element_type=F32) + bji[...])
    okj[...] = (_swish(jnp.dot(xb, Wkj[...], preferred_element_type=F32) + bkj[...])
                * jnp.dot(rbf[...], lr[...], preferred_element_type=F32))


def _pre(x, rbf, Wji, bji, Wkj, bkj, lr):
    return pl.pallas_call(
        _pre_body,
        grid=(E_GRID,),
        in_specs=[_rows(H), _rows(6), _full((H, H)), _full((1, H)),
                  _full((H, H)), _full((1, H)), _full((6, H))],
        out_specs=[_rows(H), _rows(H)],
        out_shape=[jax.ShapeDtypeStruct((N_EDGES, H), F32),
                   jax.ShapeDtypeStruct((N_EDGES, H), F32)],
    )(x, rbf, Wji, bji, Wkj, bkj, lr)


def _tri_body(g, sbf, ls, U8, out):
    sb = jnp.dot(sbf[...], ls[...], preferred_element_type=F32)
    gb = g[...]
    acc = jnp.zeros((BLK, H), F32)
    for jj in range(8):
        acc = acc + sb[:, jj:jj + 1] * jnp.dot(gb, U8[jj], preferred_element_type=F32)
    out[...] = acc


def _tri(g, sbf, ls, U8):
    return pl.pallas_call(
        _tri_body,
        grid=(E_GRID,),
        in_specs=[_rows(H), _rows(42), _full((42, 8)), _full((8, H, H))],
        out_specs=_rows(H),
        out_shape=jax.ShapeDtypeStruct((N_TRI, H), F32),
    )(g, sbf, ls, U8)


def _post_body(agg, xji, x, rbf, rb0W, rb0b, rb1W, rb1b, WlI, blI,
               q0W, q0b, q1W, q1b, q2W, q2b, q3W, q3b, orbf, ox, ot):
    h = xji[...] + agg[...]
    h1 = _swish(jnp.dot(h, rb0W[...], preferred_element_type=F32) + rb0b[...])
    h = h + _swish(jnp.dot(h1, rb1W[...], preferred_element_type=F32) + rb1b[...])
    h = _swish(jnp.dot(h, WlI[...], preferred_element_type=F32) + blI[...]) + x[...]
    h1 = _swish(jnp.dot(h, q0W[...], preferred_element_type=F32) + q0b[...])
    h = h + _swish(jnp.dot(h1, q1W[...], preferred_element_type=F32) + q1b[...])
    h1 = _swish(jnp.dot(h, q2W[...], preferred_element_type=F32) + q2b[...])
    h = h + _swish(jnp.dot(h1, q3W[...], preferred_element_type=F32) + q3b[...])
    ox[...] = h
    ot[...] = jnp.dot(rbf[...], orbf[...], preferred_element_type=F32) * h


def _post(agg, xji, x, rbf, weights):
    wspecs = [_full((H, H)), _full((1, H))] * 7
    return pl.pallas_call(
        _post_body,
        grid=(E_GRID,),
        in_specs=[_rows(H), _rows(H), _rows(H), _rows(6)] + wspecs + [_full((6, H))],
        out_specs=[_rows(H), _rows(H)],
        out_shape=[jax.ShapeDtypeStruct((N_EDGES, H), F32),
                   jax.ShapeDtypeStruct((N_EDGES, H), F32)],
    )(agg, xji, x, rbf, *weights)


def _nmlp_body(ns, W0, b0, W1, b1, W2, b2, olp, out):
    n = ns[...]
    n = _swish(jnp.dot(n, W0[...], preferred_element_type=F32) + b0[...])
    n = _swish(jnp.dot(n, W1[...], preferred_element_type=F32) + b1[...])
    n = _swish(jnp.dot(n, W2[...], preferred_element_type=F32) + b2[...])
    out[...] = jnp.dot(n, olp[...], preferred_element_type=F32)


def _nmlp(ns, W0, b0, W1, b1, W2, b2, olp):
    return pl.pallas_call(
        _nmlp_body,
        grid=(N_GRID,),
        in_specs=[_rows(H),
                  _full((H, H)), _full((1, H)), _full((H, H)), _full((1, H)),
                  _full((H, H)), _full((1, H)), _full((H, H))],
        out_specs=_rows(H),
        out_shape=jax.ShapeDtypeStruct((N_PAD, H), F32),
    )(ns, W0, b0, W1, b1, W2, b2, olp)


def _mesh():
    return plsc.VectorSubcoreMesh(core_axis_name="c", subcore_axis_name="s")


GRP = 2 * CHUNK
NB = 2


def _sc_gather(table, idx2):
    ngrp = idx2.shape[0]
    nro = ((ngrp + 31) // 32 + NB - 1) // NB

    @functools.partial(
        pl.kernel,
        out_type=jax.ShapeDtypeStruct((ngrp * GRP, H), F32),
        mesh=_mesh(),
        compiler_params=pltpu.CompilerParams(needs_layout_passes=False),
        scratch_types=[pltpu.VMEM((NB, 2, CHUNK), I32),
                       pltpu.VMEM((NB, GRP, H), F32)]
        + [pltpu.SemaphoreType.DMA] * (2 * NB),
    )
    def k(tbl, idx, out, iv, buf, *sems):
        gs, ws = sems[:NB], sems[NB:]
        wid = lax.axis_index("s") * 2 + lax.axis_index("c")
        nk = (ngrp - wid + 31) // 32

        def body(rb, carry):
            hs = {}
            for b in range(NB):
                it = rb * NB + b

                @pl.when(it < nk)
                def _fire(b=b, it=it):
                    grp = wid + it * 32

                    @pl.when(rb > 0)
                    def _drain():
                        pltpu.make_async_copy(tbl.at[pl.ds(0, GRP)],
                                              buf.at[b], ws[b]).wait()

                    pltpu.sync_copy(idx.at[grp], iv.at[b])
                    hs[b] = [pltpu.async_copy(tbl.at[iv.at[b, cc]],
                                              buf.at[b, pl.ds(cc * CHUNK, CHUNK)],
                                              gs[b])
                             for cc in range(2)]
            for b in range(NB):
                it = rb * NB + b

                @pl.when(it < nk)
                def _retire(b=b, it=it):
                    grp = wid + it * 32
                    for cp in hs[b]:
                        cp.wait()
                    pltpu.async_copy(buf.at[b], out.at[pl.ds(grp * GRP, GRP)],
                                     ws[b])
            return carry

        lax.fori_loop(0, nro, body, 0)
        for b in range(NB):

            @pl.when(nk > b)
            def _final(b=b):
                pltpu.make_async_copy(tbl.at[pl.ds(0, GRP)],
                                      buf.at[b], ws[b]).wait()

    return k(table, idx2)


def _sc_scatter_sorted(m, pexp, ldexp, blo, bhi, zrows, dr, nround, nout):
    ngexp = pexp.shape[0]
    step = dr // 16
    nwin = nout // dr

    @functools.partial(
        pl.kernel,
        out_type=jax.ShapeDtypeStruct((nout, H), F32),
        mesh=_mesh(),
        compiler_params=pltpu.CompilerParams(needs_layout_passes=False),
        scratch_types=[pltpu.VMEM((NB, 2, CHUNK), I32),
                       pltpu.VMEM((NB, 2, CHUNK), I32),
                       pltpu.VMEM((NB, GRP, H), F32),
                       pltpu.VMEM((16,), I32),
                       pltpu.VMEM((16,), I32),
                       pltpu.VMEM_SHARED((dr + 8, H), F32)]
        + [pltpu.SemaphoreType.DMA] * (2 * NB),
    )
    def k(mref, pref, ldref, blo_r, bhi_r, zr, out,
          iv, ldv, buf, blv, bhv, spm, *sems):
        gs, ss = sems[:NB], sems[NB:]
        c = lax.axis_index("c")
        sid = lax.axis_index("s")

        def window(r, carry):
            g = c * nround + r
            base = g * dr
            pltpu.sync_copy(blo_r.at[c, r], blv)
            pltpu.sync_copy(bhi_r.at[c, r], bhv)
            c0 = jnp.clip(jnp.max(blv[...]), 0, ngexp)
            c1 = jnp.clip(jnp.max(bhv[...]), 0, ngexp)

            @pl.when(g < nwin)
            def _round():
                pltpu.sync_copy(zr.at[pl.ds(0, step)],
                                spm.at[pl.ds(sid * step, step)])
                plsc.subcore_barrier()
                nk = jnp.maximum(0, (c1 - c0 - sid + 15) // 16)
                nro = (nk + NB - 1) // NB

                def body(rb, carry):
                    hs = {}
                    for b in range(NB):
                        it = rb * NB + b

                        @pl.when(it < nk)
                        def _fire(b=b, it=it):
                            grp = jnp.clip(c0 + sid + it * 16, 0, ngexp - 1)

                            @pl.when(rb > 0)
                            def _drain():
                                pltpu.make_async_copy(mref.at[pl.ds(0, GRP)],
                                                      buf.at[b], ss[b]).wait()

                            pltpu.sync_copy(pref.at[grp], iv.at[b])
                            pltpu.sync_copy(ldref.at[grp], ldv.at[b])
                            hs[b] = [pltpu.async_copy(
                                mref.at[iv.at[b, cc]],
                                buf.at[b, pl.ds(cc * CHUNK, CHUNK)], gs[b])
                                for cc in range(2)]
                    for b in range(NB):
                        it = rb * NB + b

                        @pl.when(it < nk)
                        def _retire(b=b, it=it):
                            for cp in hs[b]:
                                cp.wait()
                            for cc in range(2):
                                pltpu.async_copy(
                                    buf.at[b, pl.ds(cc * CHUNK, CHUNK)],
                                    spm.at[ldv.at[b, cc]], ss[b], add=True)
                    return carry

                lax.fori_loop(0, nro, body, 0)
                for b in range(NB):

                    @pl.when(nk > b)
                    def _final(b=b):
                        pltpu.make_async_copy(mref.at[pl.ds(0, GRP)],
                                              buf.at[b], ss[b]).wait()

                plsc.subcore_barrier()
                pltpu.sync_copy(spm.at[pl.ds(sid * step, step)],
                                out.at[pl.ds(base + sid * step, step)])

            return carry

        lax.fori_loop(0, nround, window, 0)

    return k(m, pexp, ldexp, blo, bhi, zrows)


def _scatter_plan(dest, dr, nwin, nround):
    n = dest.shape[0]
    perm = jnp.argsort(dest).astype(I32)
    s = jnp.take(dest, perm)
    bounds = jnp.searchsorted(s, jnp.arange(nwin + 1) * dr).astype(I32)
    nwin_sz = bounds[1:] - bounds[:-1]
    npad = ((nwin_sz + GRP - 1) // GRP) * GRP
    start_p = jnp.concatenate([jnp.zeros((1,), I32), jnp.cumsum(npad).astype(I32)])
    w_of = s // dr
    pos = jnp.arange(n, dtype=I32) - bounds[w_of] + start_p[w_of]
    ngexp = n // GRP + nwin
    texp = ngexp * GRP
    pexp = jnp.zeros((texp,), I32).at[pos].set(perm).reshape(ngexp, 2, CHUNK)
    ldexp = jnp.full((texp,), dr, I32).at[pos].set(s % dr).reshape(ngexp, 2, CHUNK)
    cb = start_p // GRP

    def pack(a, b):
        v = jnp.stack([a, b])
        return jnp.broadcast_to(v[:, :, None], (2, nround, 16))

    blo = pack(cb[0:nround], cb[nround:nwin])
    bhi = pack(cb[1:nround + 1], cb[nround + 1:nwin + 1])
    return pexp, ldexp, blo, bhi


def kernel(z, rbf, sbf, i, j, idx_kj, idx_ji, params):
    p = params
    i = i.astype(I32)
    j = j.astype(I32)
    idx_kj = idx_kj.astype(I32)
    idx_ji = idx_ji.astype(I32)

    z3 = jnp.pad(z.astype(I32), (0, N_PAD - N_NODES)).reshape(N_GRID, 1, BLK)
    ij2 = jnp.concatenate([i, j]).reshape((2 * N_EDGES) // GRP, 2, CHUNK)
    kj2 = idx_kj.reshape(N_TRI // GRP, 2, CHUNK)
    peE, ldE, bloE, bhiE = _scatter_plan(idx_ji, DR, NWIN, NROUND)
    peN, ldN, bloN, bhiN = _scatter_plan(i, DRN, NWINN, 1)
    zrows = jnp.zeros((DR // 16, H), F32)

    def b1(v):
        return v.reshape(1, H)

    W1 = p['emb_lin_W'][0:H]
    W2 = p['emb_lin_W'][H:2 * H]
    W3 = p['emb_lin_W'][2 * H:3 * H]

    xn = _node_embed(z3, p['emb'])
    xij = _sc_gather(xn, ij2)
    xi = xij[:N_EDGES]
    xj = xij[N_EDGES:]
    x, t = _embed(xi, xj, rbf, p['emb_lin_rbf_W'], b1(p['emb_lin_rbf_b']),
                  W1, W2, W3, b1(p['emb_lin_b']), p['out_lin_rbf'][0])

    def out_block(bb, t_e):
        ns = _sc_scatter_sorted(t_e, peN, ldN, bloN, bhiN, zrows, DRN, 1, N_PAD)
        olp = jnp.pad(p['out_lin'][bb], ((0, 0), (0, H - p['out_lin'][bb].shape[1])))
        return _nmlp(ns, p['out_lins_W'][bb, 0], b1(p['out_lins_b'][bb, 0]),
                     p['out_lins_W'][bb, 1], b1(p['out_lins_b'][bb, 1]),
                     p['out_lins_W'][bb, 2], b1(p['out_lins_b'][bb, 2]), olp)

    P = out_block(0, t)
    for b in range(6):
        xji, xkj = _pre(x, rbf, p['int_lin_ji_W'][b], b1(p['int_lin_ji_b'][b]),
                        p['int_lin_kj_W'][b], b1(p['int_lin_kj_b'][b]),
                        p['int_lin_rbf'][b])
        g = _sc_gather(xkj, kj2)
        U8 = jnp.transpose(p['int_W'][b], (1, 2, 0))
        m = _tri(g, sbf, p['int_lin_sbf'][b], U8)
        agg = _sc_scatter_sorted(m, peE, ldE, bloE, bhiE, zrows, DR, NROUND, N_EDGES)
        weights = [
            p['res_before_W'][b, 0, 0], b1(p['res_before_b'][b, 0, 0]),
            p['res_before_W'][b, 0, 1], b1(p['res_before_b'][b, 0, 1]),
            p['int_lin_W'][b], b1(p['int_lin_b'][b]),
            p['res_after_W'][b, 0, 0], b1(p['res_after_b'][b, 0, 0]),
            p['res_after_W'][b, 0, 1], b1(p['res_after_b'][b, 0, 1]),
            p['res_after_W'][b, 1, 0], b1(p['res_after_b'][b, 1, 0]),
            p['res_after_W'][b, 1, 1], b1(p['res_after_b'][b, 1, 1]),
            p['out_lin_rbf'][b + 1],
        ]
        x, t = _post(agg, xji, x, rbf, weights)
        P = P + out_block(b + 1, t)
    return P[:N_NODES, 0:1]


# speedup vs baseline: 1.1574x; 1.0976x over previous
import functools

import jax
import jax.numpy as jnp
from jax import lax
from jax.experimental import pallas as pl
from jax.experimental.pallas import tpu as pltpu
from jax.experimental.pallas import tpu_sc as plsc

F32 = jnp.float32
I32 = jnp.int32

N_NODES = 10000
N_PAD = 10240
N_EDGES = 320000
N_TRI = 320000
H = 128
BLK = 512
E_GRID = N_EDGES // BLK
N_GRID = N_PAD // BLK
CHUNK = 128
DR = 6400
NWIN = N_EDGES // DR
NROUND = 25
DRN = 5120
NWINN = N_PAD // DRN


def _swish(v):
    return v * jax.nn.sigmoid(v)


def _rows(cols):
    return pl.BlockSpec((BLK, cols), lambda ii: (ii, 0))


def _full(shape):
    nd = len(shape)
    return pl.BlockSpec(shape, lambda ii: (0,) * nd)


def _nodeemb_body(z_ref, emb_ref, out_ref):
    zb = z_ref[0, 0]
    oh = (zb[:, None] == lax.broadcasted_iota(I32, (1, 95), 1)).astype(F32)
    out_ref[...] = jnp.dot(oh, emb_ref[...], preferred_element_type=F32)


def _node_embed(z3, emb):
    return pl.pallas_call(
        _nodeemb_body,
        grid=(N_GRID,),
        in_specs=[pl.BlockSpec((1, 1, BLK), lambda ii: (ii, 0, 0)), _full((95, H))],
        out_specs=_rows(H),
        out_shape=jax.ShapeDtypeStruct((N_PAD, H), F32),
    )(z3, emb)


def _embed_body(xi, xj, rbf, Wr, br, W1, W2, W3, be, orbf, ox, ot):
    rb = rbf[...]
    re = _swish(jnp.dot(rb, Wr[...], preferred_element_type=F32) + br[...])
    acc = (jnp.dot(xi[...], W1[...], preferred_element_type=F32)
           + jnp.dot(xj[...], W2[...], preferred_element_type=F32)
           + jnp.dot(re, W3[...], preferred_element_type=F32) + be[...])
    xv = _swish(acc)
    ox[...] = xv
    ot[...] = jnp.dot(rb, orbf[...], preferred_element_type=F32) * xv


def _embed(xi, xj, rbf, Wr, br, W1, W2, W3, be, orbf):
    return pl.pallas_call(
        _embed_body,
        grid=(E_GRID,),
        in_specs=[_rows(H), _rows(H), _rows(6), _full((6, H)), _full((1, H)),
                  _full((H, H)), _full((H, H)), _full((H, H)), _full((1, H)),
                  _full((6, H))],
        out_specs=[_rows(H), _rows(H)],
        out_shape=[jax.ShapeDtypeStruct((N_EDGES, H), F32),
                   jax.ShapeDtypeStruct((N_EDGES, H), F32)],
    )(xi, xj, rbf, Wr, br, W1, W2, W3, be, orbf)


def _pre_body(x, rbf, Wji, bji, Wkj, bkj, lr, oji, okj):
    xb = x[...]
    oji[...] = _swish(jnp.dot(xb, Wji[...], preferred_element_type=F32) + bji[...])
    okj[...] = (_swish(jnp.dot(xb, Wkj[...], preferred_element_type=F32) + bkj[...])
                * jnp.dot(rbf[...], lr[...], preferred_element_type=F32))


def _pre(x, rbf, Wji, bji, Wkj, bkj, lr):
    return pl.pallas_call(
        _pre_body,
        grid=(E_GRID,),
        in_specs=[_rows(H), _rows(6), _full((H, H)), _full((1, H)),
                  _full((H, H)), _full((1, H)), _full((6, H))],
        out_specs=[_rows(H), _rows(H)],
        out_shape=[jax.ShapeDtypeStruct((N_EDGES, H), F32),
                   jax.ShapeDtypeStruct((N_EDGES, H), F32)],
    )(x, rbf, Wji, bji, Wkj, bkj, lr)


def _tri_body(g, sbf, ls, U8, out):
    sb = jnp.dot(sbf[...], ls[...], preferred_element_type=F32)
    gb = g[...]
    acc = jnp.zeros((BLK, H), F32)
    for jj in range(8):
        acc = acc + sb[:, jj:jj + 1] * jnp.dot(gb, U8[jj], preferred_element_type=F32)
    out[...] = acc


def _tri(g, sbf, ls, U8):
    return pl.pallas_call(
        _tri_body,
        grid=(E_GRID,),
        in_specs=[_rows(H), _rows(42), _full((42, 8)), _full((8, H, H))],
        out_specs=_rows(H),
        out_shape=jax.ShapeDtypeStruct((N_TRI, H), F32),
    )(g, sbf, ls, U8)


def _post_body(agg, xji, x, rbf, rb0W, rb0b, rb1W, rb1b, WlI, blI,
               q0W, q0b, q1W, q1b, q2W, q2b, q3W, q3b, orbf, ox, ot):
    h = xji[...] + agg[...]
    h1 = _swish(jnp.dot(h, rb0W[...], preferred_element_type=F32) + rb0b[...])
    h = h + _swish(jnp.dot(h1, rb1W[...], preferred_element_type=F32) + rb1b[...])
    h = _swish(jnp.dot(h, WlI[...], preferred_element_type=F32) + blI[...]) + x[...]
    h1 = _swish(jnp.dot(h, q0W[...], preferred_element_type=F32) + q0b[...])
    h = h + _swish(jnp.dot(h1, q1W[...], preferred_element_type=F32) + q1b[...])
    h1 = _swish(jnp.dot(h, q2W[...], preferred_element_type=F32) + q2b[...])
    h = h + _swish(jnp.dot(h1, q3W[...], preferred_element_type=F32) + q3b[...])
    ox[...] = h
    ot[...] = jnp.dot(rbf[...], orbf[...], preferred_element_type=F32) * h


def _post(agg, xji, x, rbf, weights):
    wspecs = [_full((H, H)), _full((1, H))] * 7
    return pl.pallas_call(
        _post_body,
        grid=(E_GRID,),
        in_specs=[_rows(H), _rows(H), _rows(H), _rows(6)] + wspecs + [_full((6, H))],
        out_specs=[_rows(H), _rows(H)],
        out_shape=[jax.ShapeDtypeStruct((N_EDGES, H), F32),
                   jax.ShapeDtypeStruct((N_EDGES, H), F32)],
    )(agg, xji, x, rbf, *weights)


def _nmlp_body(ns, W0, b0, W1, b1, W2, b2, olp, out):
    n = ns[0] + ns[1]
    n = _swish(jnp.dot(n, W0[...], preferred_element_type=F32) + b0[...])
    n = _swish(jnp.dot(n, W1[...], preferred_element_type=F32) + b1[...])
    n = _swish(jnp.dot(n, W2[...], preferred_element_type=F32) + b2[...])
    out[...] = jnp.dot(n, olp[...], preferred_element_type=F32)


def _nmlp(ns, W0, b0, W1, b1, W2, b2, olp):
    return pl.pallas_call(
        _nmlp_body,
        grid=(N_GRID,),
        in_specs=[pl.BlockSpec((2, BLK, H), lambda ii: (0, ii, 0)),
                  _full((H, H)), _full((1, H)), _full((H, H)), _full((1, H)),
                  _full((H, H)), _full((1, H)), _full((H, H))],
        out_specs=_rows(H),
        out_shape=jax.ShapeDtypeStruct((N_PAD, H), F32),
    )(ns, W0, b0, W1, b1, W2, b2, olp)


def _mesh():
    return plsc.VectorSubcoreMesh(core_axis_name="c", subcore_axis_name="s")


GRP = 2 * CHUNK
NB = 2


def _sc_gather(table, idx2):
    ngrp = idx2.shape[0]
    nro = ((ngrp + 31) // 32 + NB - 1) // NB

    @functools.partial(
        pl.kernel,
        out_type=jax.ShapeDtypeStruct((ngrp * GRP, H), F32),
        mesh=_mesh(),
        compiler_params=pltpu.CompilerParams(needs_layout_passes=False),
        scratch_types=[pltpu.VMEM((NB, 2, CHUNK), I32),
                       pltpu.VMEM((NB, GRP, H), F32)]
        + [pltpu.SemaphoreType.DMA] * (2 * NB),
    )
    def k(tbl, idx, out, iv, buf, *sems):
        gs, ws = sems[:NB], sems[NB:]
        wid = lax.axis_index("s") * 2 + lax.axis_index("c")
        nk = (ngrp - wid + 31) // 32

        def body(rb, carry):
            hs = {}
            for b in range(NB):
                it = rb * NB + b

                @pl.when(it < nk)
                def _fire(b=b, it=it):
                    grp = wid + it * 32

                    @pl.when(rb > 0)
                    def _drain():
                        pltpu.make_async_copy(tbl.at[pl.ds(0, GRP)],
                                              buf.at[b], ws[b]).wait()

                    pltpu.sync_copy(idx.at[grp], iv.at[b])
                    hs[b] = [pltpu.async_copy(tbl.at[iv.at[b, cc]],
                                              buf.at[b, pl.ds(cc * CHUNK, CHUNK)],
                                              gs[b])
                             for cc in range(2)]
            for b in range(NB):
                it = rb * NB + b

                @pl.when(it < nk)
                def _retire(b=b, it=it):
                    grp = wid + it * 32
                    for cp in hs[b]:
                        cp.wait()
                    pltpu.async_copy(buf.at[b], out.at[pl.ds(grp * GRP, GRP)],
                                     ws[b])
            return carry

        lax.fori_loop(0, nro, body, 0)
        for b in range(NB):

            @pl.when(nk > b)
            def _final(b=b):
                pltpu.make_async_copy(tbl.at[pl.ds(0, GRP)],
                                      buf.at[b], ws[b]).wait()

    return k(table, idx2)


def _sc_scatter_acc(vals, i2, zrows):
    ngrp = N_EDGES // CHUNK
    nro = ((ngrp + 31) // 32 + NB - 1) // NB

    @functools.partial(
        pl.kernel,
        out_type=jax.ShapeDtypeStruct((2, N_PAD, H), F32),
        mesh=_mesh(),
        compiler_params=pltpu.CompilerParams(needs_layout_passes=False),
        scratch_types=[pltpu.VMEM((NB, 1, CHUNK), I32),
                       pltpu.VMEM((NB, CHUNK, H), F32),
                       pltpu.VMEM_SHARED((N_PAD, H), F32)]
        + [pltpu.SemaphoreType.DMA] * (2 * NB),
    )
    def k(vref, idx, zr, out, iv, buf, spm, *sems):
        gs, ss = sems[:NB], sems[NB:]
        c = lax.axis_index("c")
        sid = lax.axis_index("s")
        wid = sid * 2 + c
        pltpu.sync_copy(zr.at[pl.ds(0, 640)], spm.at[pl.ds(sid * 640, 640)])
        plsc.subcore_barrier()
        nk = (ngrp - wid + 31) // 32

        def body(rb, carry):
            hs = {}
            for b in range(NB):
                it = rb * NB + b
                grp = wid + it * 32
                hs[b] = pltpu.make_async_copy(
                    vref.at[pl.ds(grp * CHUNK, CHUNK)], buf.at[b], gs[b])

                @pl.when(it < nk)
                def _fire(b=b, it=it, grp=grp):

                    @pl.when(rb > 0)
                    def _drain():
                        pltpu.make_async_copy(vref.at[pl.ds(0, CHUNK)],
                                              buf.at[b], ss[b]).wait()

                    pltpu.sync_copy(idx.at[grp], iv.at[b])
                    pltpu.async_copy(vref.at[pl.ds(grp * CHUNK, CHUNK)],
                                     buf.at[b], gs[b])
            for b in range(NB):
                it = rb * NB + b

                @pl.when(it < nk)
                def _retire(b=b, it=it):
                    hs[b].wait()
                    pltpu.async_copy(buf.at[b], spm.at[iv.at[b, 0]],
                                     ss[b], add=True)
            return carry

        lax.fori_loop(0, nro, body, 0)
        for b in range(NB):

            @pl.when(nk > b)
            def _final(b=b):
                pltpu.make_async_copy(vref.at[pl.ds(0, CHUNK)],
                                      buf.at[b], ss[b]).wait()

        plsc.subcore_barrier()
        pltpu.sync_copy(spm.at[pl.ds(sid * 640, 640)],
                        out.at[c, pl.ds(sid * 640, 640)])

    return k(vals, i2, zrows)


def _sc_scatter_sorted(m, pexp, ldexp, blo, bhi, zrows, dr, nround, nout):
    ngexp = pexp.shape[0]
    step = dr // 16
    nwin = nout // dr

    @functools.partial(
        pl.kernel,
        out_type=jax.ShapeDtypeStruct((nout, H), F32),
        mesh=_mesh(),
        compiler_params=pltpu.CompilerParams(needs_layout_passes=False),
        scratch_types=[pltpu.VMEM((NB, 2, CHUNK), I32),
                       pltpu.VMEM((NB, 2, CHUNK), I32),
                       pltpu.VMEM((NB, GRP, H), F32),
                       pltpu.VMEM((16,), I32),
                       pltpu.VMEM((16,), I32),
                       pltpu.VMEM_SHARED((dr + 8, H), F32)]
        + [pltpu.SemaphoreType.DMA] * (2 * NB),
    )
    def k(mref, pref, ldref, blo_r, bhi_r, zr, out,
          iv, ldv, buf, blv, bhv, spm, *sems):
        gs, ss = sems[:NB], sems[NB:]
        c = lax.axis_index("c")
        sid = lax.axis_index("s")

        def window(r, carry):
            g = c * nround + r
            base = g * dr
            pltpu.sync_copy(blo_r.at[c, r], blv)
            pltpu.sync_copy(bhi_r.at[c, r], bhv)
            c0 = jnp.clip(jnp.max(blv[...]), 0, ngexp)
            c1 = jnp.clip(jnp.max(bhv[...]), 0, ngexp)

            @pl.when(g < nwin)
            def _round():
                pltpu.sync_copy(zr.at[pl.ds(0, step)],
                                spm.at[pl.ds(sid * step, step)])
                plsc.subcore_barrier()
                nk = jnp.maximum(0, (c1 - c0 - sid + 15) // 16)
                nro = (nk + NB - 1) // NB

                def body(rb, carry):
                    hs = {}
                    for b in range(NB):
                        it = rb * NB + b

                        @pl.when(it < nk)
                        def _fire(b=b, it=it):
                            grp = jnp.clip(c0 + sid + it * 16, 0, ngexp - 1)

                            @pl.when(rb > 0)
                            def _drain():
                                pltpu.make_async_copy(mref.at[pl.ds(0, GRP)],
                                                      buf.at[b], ss[b]).wait()

                            pltpu.sync_copy(pref.at[grp], iv.at[b])
                            pltpu.sync_copy(ldref.at[grp], ldv.at[b])
                            hs[b] = [pltpu.async_copy(
                                mref.at[iv.at[b, cc]],
                                buf.at[b, pl.ds(cc * CHUNK, CHUNK)], gs[b])
                                for cc in range(2)]
                    for b in range(NB):
                        it = rb * NB + b

                        @pl.when(it < nk)
                        def _retire(b=b, it=it):
                            for cp in hs[b]:
                                cp.wait()
                            for cc in range(2):
                                pltpu.async_copy(
                                    buf.at[b, pl.ds(cc * CHUNK, CHUNK)],
                                    spm.at[ldv.at[b, cc]], ss[b], add=True)
                    return carry

                lax.fori_loop(0, nro, body, 0)
                for b in range(NB):

                    @pl.when(nk > b)
                    def _final(b=b):
                        pltpu.make_async_copy(mref.at[pl.ds(0, GRP)],
                                              buf.at[b], ss[b]).wait()

                plsc.subcore_barrier()
                pltpu.sync_copy(spm.at[pl.ds(sid * step, step)],
                                out.at[pl.ds(base + sid * step, step)])

            return carry

        lax.fori_loop(0, nround, window, 0)

    return k(m, pexp, ldexp, blo, bhi, zrows)


def _scatter_plan(dest, dr, nwin, nround):
    n = dest.shape[0]
    perm = jnp.argsort(dest).astype(I32)
    s = jnp.take(dest, perm)
    bounds = jnp.searchsorted(s, jnp.arange(nwin + 1) * dr).astype(I32)
    nwin_sz = bounds[1:] - bounds[:-1]
    npad = ((nwin_sz + GRP - 1) // GRP) * GRP
    start_p = jnp.concatenate([jnp.zeros((1,), I32), jnp.cumsum(npad).astype(I32)])
    w_of = s // dr
    pos = jnp.arange(n, dtype=I32) - bounds[w_of] + start_p[w_of]
    ngexp = n // GRP + nwin
    texp = ngexp * GRP
    pexp = jnp.zeros((texp,), I32).at[pos].set(perm).reshape(ngexp, 2, CHUNK)
    ldexp = jnp.full((texp,), dr, I32).at[pos].set(s % dr).reshape(ngexp, 2, CHUNK)
    cb = start_p // GRP

    def pack(a, b):
        v = jnp.stack([a, b])
        return jnp.broadcast_to(v[:, :, None], (2, nround, 16))

    blo = pack(cb[0:nround], cb[nround:nwin])
    bhi = pack(cb[1:nround + 1], cb[nround + 1:nwin + 1])
    return pexp, ldexp, blo, bhi


def kernel(z, rbf, sbf, i, j, idx_kj, idx_ji, params):
    p = params
    i = i.astype(I32)
    j = j.astype(I32)
    idx_kj = idx_kj.astype(I32)
    idx_ji = idx_ji.astype(I32)

    z3 = jnp.pad(z.astype(I32), (0, N_PAD - N_NODES)).reshape(N_GRID, 1, BLK)
    ij2 = jnp.concatenate([i, j]).reshape((2 * N_EDGES) // GRP, 2, CHUNK)
    kj2 = idx_kj.reshape(N_TRI // GRP, 2, CHUNK)
    i2 = i.reshape(N_EDGES // CHUNK, 1, CHUNK)
    peE, ldE, bloE, bhiE = _scatter_plan(idx_ji, DR, NWIN, NROUND)
    zrows = jnp.zeros((N_PAD // 16, H), F32)

    def b1(v):
        return v.reshape(1, H)

    W1 = p['emb_lin_W'][0:H]
    W2 = p['emb_lin_W'][H:2 * H]
    W3 = p['emb_lin_W'][2 * H:3 * H]

    xn = _node_embed(z3, p['emb'])
    xij = _sc_gather(xn, ij2)
    xi = xij[:N_EDGES]
    xj = xij[N_EDGES:]
    x, t = _embed(xi, xj, rbf, p['emb_lin_rbf_W'], b1(p['emb_lin_rbf_b']),
                  W1, W2, W3, b1(p['emb_lin_b']), p['out_lin_rbf'][0])

    def out_block(bb, t_e):
        ns = _sc_scatter_acc(t_e, i2, zrows)
        olp = jnp.pad(p['out_lin'][bb], ((0, 0), (0, H - p['out_lin'][bb].shape[1])))
        return _nmlp(ns, p['out_lins_W'][bb, 0], b1(p['out_lins_b'][bb, 0]),
                     p['out_lins_W'][bb, 1], b1(p['out_lins_b'][bb, 1]),
                     p['out_lins_W'][bb, 2], b1(p['out_lins_b'][bb, 2]), olp)

    P = out_block(0, t)
    for b in range(6):
        xji, xkj = _pre(x, rbf, p['int_lin_ji_W'][b], b1(p['int_lin_ji_b'][b]),
                        p['int_lin_kj_W'][b], b1(p['int_lin_kj_b'][b]),
                        p['int_lin_rbf'][b])
        g = _sc_gather(xkj, kj2)
        U8 = jnp.transpose(p['int_W'][b], (1, 2, 0))
        m = _tri(g, sbf, p['int_lin_sbf'][b], U8)
        agg = _sc_scatter_sorted(m, peE, ldE, bloE, bhiE, zrows, DR, NROUND, N_EDGES)
        weights = [
            p['res_before_W'][b, 0, 0], b1(p['res_before_b'][b, 0, 0]),
            p['res_before_W'][b, 0, 1], b1(p['res_before_b'][b, 0, 1]),
            p['int_lin_W'][b], b1(p['int_lin_b'][b]),
            p['res_after_W'][b, 0, 0], b1(p['res_after_b'][b, 0, 0]),
            p['res_after_W'][b, 0, 1], b1(p['res_after_b'][b, 0, 1]),
            p['res_after_W'][b, 1, 0], b1(p['res_after_b'][b, 1, 0]),
            p['res_after_W'][b, 1, 1], b1(p['res_after_b'][b, 1, 1]),
            p['out_lin_rbf'][b + 1],
        ]
        x, t = _post(agg, xji, x, rbf, weights)
        P = P + out_block(b + 1, t)
    return P[:N_NODES, 0:1]


# speedup vs baseline: 1.1587x; 1.0012x over previous
import functools

import jax
import jax.numpy as jnp
from jax import lax
from jax.experimental import pallas as pl
from jax.experimental.pallas import tpu as pltpu
from jax.experimental.pallas import tpu_sc as plsc

F32 = jnp.float32
I32 = jnp.int32

N_NODES = 10000
N_PAD = 10240
N_EDGES = 320000
N_TRI = 320000
H = 128
BLK = 512
E_GRID = N_EDGES // BLK
N_GRID = N_PAD // BLK
CHUNK = 128
DR = 6400
NWIN = N_EDGES // DR
NROUND = 25
DRN = 5120
NWINN = N_PAD // DRN


def _swish(v):
    return v * jax.nn.sigmoid(v)


def _rows(cols):
    return pl.BlockSpec((BLK, cols), lambda ii: (ii, 0))


def _full(shape):
    nd = len(shape)
    return pl.BlockSpec(shape, lambda ii: (0,) * nd)


def _nodeemb_body(z_ref, emb_ref, out_ref):
    zb = z_ref[0, 0]
    oh = (zb[:, None] == lax.broadcasted_iota(I32, (1, 95), 1)).astype(F32)
    out_ref[...] = jnp.dot(oh, emb_ref[...], preferred_element_type=F32)


def _node_embed(z3, emb):
    return pl.pallas_call(
        _nodeemb_body,
        grid=(N_GRID,),
        in_specs=[pl.BlockSpec((1, 1, BLK), lambda ii: (ii, 0, 0)), _full((95, H))],
        out_specs=_rows(H),
        out_shape=jax.ShapeDtypeStruct((N_PAD, H), F32),
    )(z3, emb)


def _embed_body(xi, xj, rbf, Wr, br, W1, W2, W3, be, orbf, ox, ot):
    rb = rbf[...]
    re = _swish(jnp.dot(rb, Wr[...], preferred_element_type=F32) + br[...])
    acc = (jnp.dot(xi[...], W1[...], preferred_element_type=F32)
           + jnp.dot(xj[...], W2[...], preferred_element_type=F32)
           + jnp.dot(re, W3[...], preferred_element_type=F32) + be[...])
    xv = _swish(acc)
    ox[...] = xv
    ot[...] = jnp.dot(rb, orbf[...], preferred_element_type=F32) * xv


def _embed(xi, xj, rbf, Wr, br, W1, W2, W3, be, orbf):
    return pl.pallas_call(
        _embed_body,
        grid=(E_GRID,),
        in_specs=[_rows(H), _rows(H), _rows(6), _full((6, H)), _full((1, H)),
                  _full((H, H)), _full((H, H)), _full((H, H)), _full((1, H)),
                  _full((6, H))],
        out_specs=[_rows(H), _rows(H)],
        out_shape=[jax.ShapeDtypeStruct((N_EDGES, H), F32),
                   jax.ShapeDtypeStruct((N_EDGES, H), F32)],
    )(xi, xj, rbf, Wr, br, W1, W2, W3, be, orbf)


def _pre_body(x, rbf, Wji, bji, Wkj, bkj, lr, oji, okj):
    xb = x[...]
    oji[...] = _swish(jnp.dot(xb, Wji[...], preferred_element_type=F32) + bji[...])
    okj[...] = (_swish(jnp.dot(xb, Wkj[...], preferred_element_type=F32) + bkj[...])
                * jnp.dot(rbf[...], lr[...], preferred_element_type=F32))


def _pre(x, rbf, Wji, bji, Wkj, bkj, lr):
    return pl.pallas_call(
        _pre_body,
        grid=(E_GRID,),
        in_specs=[_rows(H), _rows(6), _full((H, H)), _full((1, H)),
                  _full((H, H)), _full((1, H)), _full((6, H))],
        out_specs=[_rows(H), _rows(H)],
        out_shape=[jax.ShapeDtypeStruct((N_EDGES, H), F32),
                   jax.ShapeDtypeStruct((N_EDGES, H), F32)],
    )(x, rbf, Wji, bji, Wkj, bkj, lr)


def _tri_body(g, sbf, ls, U8, out):
    sb = jnp.dot(sbf[...], ls[...], preferred_element_type=F32)
    gb = g[...]
    acc = jnp.zeros((BLK, H), F32)
    for jj in range(8):
        acc = acc + sb[:, jj:jj + 1] * jnp.dot(gb, U8[jj], preferred_element_type=F32)
    out[...] = acc


def _tri(g, sbf, ls, U8):
    return pl.pallas_call(
        _tri_body,
        grid=(E_GRID,),
        in_specs=[_rows(H), _rows(42), _full((42, 8)), _full((8, H, H))],
        out_specs=_rows(H),
        out_shape=jax.ShapeDtypeStruct((N_TRI, H), F32),
    )(g, sbf, ls, U8)


def _post_body(agg, xji, x, rbf, rb0W, rb0b, rb1W, rb1b, WlI, blI,
               q0W, q0b, q1W, q1b, q2W, q2b, q3W, q3b, orbf, ox, ot):
    h = xji[...] + agg[...]
    h1 = _swish(jnp.dot(h, rb0W[...], preferred_element_type=F32) + rb0b[...])
    h = h + _swish(jnp.dot(h1, rb1W[...], preferred_element_type=F32) + rb1b[...])
    h = _swish(jnp.dot(h, WlI[...], preferred_element_type=F32) + blI[...]) + x[...]
    h1 = _swish(jnp.dot(h, q0W[...], preferred_element_type=F32) + q0b[...])
    h = h + _swish(jnp.dot(h1, q1W[...], preferred_element_type=F32) + q1b[...])
    h1 = _swish(jnp.dot(h, q2W[...], preferred_element_type=F32) + q2b[...])
    h = h + _swish(jnp.dot(h1, q3W[...], preferred_element_type=F32) + q3b[...])
    ox[...] = h
    ot[...] = jnp.dot(rbf[...], orbf[...], preferred_element_type=F32) * h


def _post(agg, xji, x, rbf, weights):
    wspecs = [_full((H, H)), _full((1, H))] * 7
    return pl.pallas_call(
        _post_body,
        grid=(E_GRID,),
        in_specs=[_rows(H), _rows(H), _rows(H), _rows(6)] + wspecs + [_full((6, H))],
        out_specs=[_rows(H), _rows(H)],
        out_shape=[jax.ShapeDtypeStruct((N_EDGES, H), F32),
                   jax.ShapeDtypeStruct((N_EDGES, H), F32)],
    )(agg, xji, x, rbf, *weights)


def _nmlp_body(ns, W0, b0, W1, b1, W2, b2, olp, out):
    n = ns[0] + ns[1]
    n = _swish(jnp.dot(n, W0[...], preferred_element_type=F32) + b0[...])
    n = _swish(jnp.dot(n, W1[...], preferred_element_type=F32) + b1[...])
    n = _swish(jnp.dot(n, W2[...], preferred_element_type=F32) + b2[...])
    out[...] = jnp.dot(n, olp[...], preferred_element_type=F32)


def _nmlp(ns, W0, b0, W1, b1, W2, b2, olp):
    return pl.pallas_call(
        _nmlp_body,
        grid=(N_GRID,),
        in_specs=[pl.BlockSpec((2, BLK, H), lambda ii: (0, ii, 0)),
                  _full((H, H)), _full((1, H)), _full((H, H)), _full((1, H)),
                  _full((H, H)), _full((1, H)), _full((H, H))],
        out_specs=_rows(H),
        out_shape=jax.ShapeDtypeStruct((N_PAD, H), F32),
    )(ns, W0, b0, W1, b1, W2, b2, olp)


def _mesh():
    return plsc.VectorSubcoreMesh(core_axis_name="c", subcore_axis_name="s")


GRP = 2 * CHUNK
NB = 2


def _sc_gather(table, idx2):
    NBG = 3
    ngrp = idx2.shape[0]
    nro = ((ngrp + 31) // 32 + NBG - 1) // NBG

    @functools.partial(
        pl.kernel,
        out_type=jax.ShapeDtypeStruct((ngrp * GRP, H), F32),
        mesh=_mesh(),
        compiler_params=pltpu.CompilerParams(needs_layout_passes=False),
        scratch_types=[pltpu.VMEM((NBG, 2, CHUNK), I32),
                       pltpu.VMEM((NBG, GRP, H), F32)]
        + [pltpu.SemaphoreType.DMA] * (2 * NBG),
    )
    def k(tbl, idx, out, iv, buf, *sems):
        gs, ws = sems[:NBG], sems[NBG:]
        wid = lax.axis_index("s") * 2 + lax.axis_index("c")
        nk = (ngrp - wid + 31) // 32

        def body(rb, carry):
            hs = {}
            for b in range(NBG):
                it = rb * NBG + b

                @pl.when(it < nk)
                def _fire(b=b, it=it):
                    grp = wid + it * 32

                    @pl.when(rb > 0)
                    def _drain():
                        pltpu.make_async_copy(tbl.at[pl.ds(0, GRP)],
                                              buf.at[b], ws[b]).wait()

                    pltpu.sync_copy(idx.at[grp], iv.at[b])
                    hs[b] = [pltpu.async_copy(tbl.at[iv.at[b, cc]],
                                              buf.at[b, pl.ds(cc * CHUNK, CHUNK)],
                                              gs[b])
                             for cc in range(2)]
            for b in range(NBG):
                it = rb * NBG + b

                @pl.when(it < nk)
                def _retire(b=b, it=it):
                    grp = wid + it * 32
                    for cp in hs[b]:
                        cp.wait()
                    pltpu.async_copy(buf.at[b], out.at[pl.ds(grp * GRP, GRP)],
                                     ws[b])
            return carry

        lax.fori_loop(0, nro, body, 0)
        for b in range(NBG):

            @pl.when(nk > b)
            def _final(b=b):
                pltpu.make_async_copy(tbl.at[pl.ds(0, GRP)],
                                      buf.at[b], ws[b]).wait()

    return k(table, idx2)


def _sc_scatter_acc(vals, i2, zrows):
    ngrp = N_EDGES // CHUNK
    nro = ((ngrp + 31) // 32 + NB - 1) // NB

    @functools.partial(
        pl.kernel,
        out_type=jax.ShapeDtypeStruct((2, N_PAD, H), F32),
        mesh=_mesh(),
        compiler_params=pltpu.CompilerParams(needs_layout_passes=False),
        scratch_types=[pltpu.VMEM((NB, 1, CHUNK), I32),
                       pltpu.VMEM((NB, CHUNK, H), F32),
                       pltpu.VMEM_SHARED((N_PAD, H), F32)]
        + [pltpu.SemaphoreType.DMA] * (2 * NB),
    )
    def k(vref, idx, zr, out, iv, buf, spm, *sems):
        gs, ss = sems[:NB], sems[NB:]
        c = lax.axis_index("c")
        sid = lax.axis_index("s")
        wid = sid * 2 + c
        pltpu.sync_copy(zr.at[pl.ds(0, 640)], spm.at[pl.ds(sid * 640, 640)])
        plsc.subcore_barrier()
        nk = (ngrp - wid + 31) // 32

        def body(rb, carry):
            hs = {}
            for b in range(NB):
                it = rb * NB + b
                grp = wid + it * 32
                hs[b] = pltpu.make_async_copy(
                    vref.at[pl.ds(grp * CHUNK, CHUNK)], buf.at[b], gs[b])

                @pl.when(it < nk)
                def _fire(b=b, it=it, grp=grp):

                    @pl.when(rb > 0)
                    def _drain():
                        pltpu.make_async_copy(vref.at[pl.ds(0, CHUNK)],
                                              buf.at[b], ss[b]).wait()

                    pltpu.sync_copy(idx.at[grp], iv.at[b])
                    pltpu.async_copy(vref.at[pl.ds(grp * CHUNK, CHUNK)],
                                     buf.at[b], gs[b])
            for b in range(NB):
                it = rb * NB + b

                @pl.when(it < nk)
                def _retire(b=b, it=it):
                    hs[b].wait()
                    pltpu.async_copy(buf.at[b], spm.at[iv.at[b, 0]],
                                     ss[b], add=True)
            return carry

        lax.fori_loop(0, nro, body, 0)
        for b in range(NB):

            @pl.when(nk > b)
            def _final(b=b):
                pltpu.make_async_copy(vref.at[pl.ds(0, CHUNK)],
                                      buf.at[b], ss[b]).wait()

        plsc.subcore_barrier()
        pltpu.sync_copy(spm.at[pl.ds(sid * 640, 640)],
                        out.at[c, pl.ds(sid * 640, 640)])

    return k(vals, i2, zrows)


def _sc_scatter_sorted(m, pexp, ldexp, blo, bhi, zrows, dr, nround, nout):
    ngexp = pexp.shape[0]
    step = dr // 16
    nwin = nout // dr

    @functools.partial(
        pl.kernel,
        out_type=jax.ShapeDtypeStruct((nout, H), F32),
        mesh=_mesh(),
        compiler_params=pltpu.CompilerParams(needs_layout_passes=False),
        scratch_types=[pltpu.VMEM((NB, 2, CHUNK), I32),
                       pltpu.VMEM((NB, 2, CHUNK), I32),
                       pltpu.VMEM((NB, GRP, H), F32),
                       pltpu.VMEM((16,), I32),
                       pltpu.VMEM((16,), I32),
                       pltpu.VMEM_SHARED((dr + 8, H), F32)]
        + [pltpu.SemaphoreType.DMA] * (2 * NB),
    )
    def k(mref, pref, ldref, blo_r, bhi_r, zr, out,
          iv, ldv, buf, blv, bhv, spm, *sems):
        gs, ss = sems[:NB], sems[NB:]
        c = lax.axis_index("c")
        sid = lax.axis_index("s")

        def window(r, carry):
            g = c * nround + r
            base = g * dr
            pltpu.sync_copy(blo_r.at[c, r], blv)
            pltpu.sync_copy(bhi_r.at[c, r], bhv)
            c0 = jnp.clip(jnp.max(blv[...]), 0, ngexp)
            c1 = jnp.clip(jnp.max(bhv[...]), 0, ngexp)

            @pl.when(g < nwin)
            def _round():
                pltpu.sync_copy(zr.at[pl.ds(0, step)],
                                spm.at[pl.ds(sid * step, step)])
                plsc.subcore_barrier()
                nk = jnp.maximum(0, (c1 - c0 - sid + 15) // 16)
                nro = (nk + NB - 1) // NB

                def body(rb, carry):
                    hs = {}
                    for b in range(NB):
                        it = rb * NB + b

                        @pl.when(it < nk)
                        def _fire(b=b, it=it):
                            grp = jnp.clip(c0 + sid + it * 16, 0, ngexp - 1)

                            @pl.when(rb > 0)
                            def _drain():
                                pltpu.make_async_copy(mref.at[pl.ds(0, GRP)],
                                                      buf.at[b], ss[b]).wait()

                            pltpu.sync_copy(pref.at[grp], iv.at[b])
                            pltpu.sync_copy(ldref.at[grp], ldv.at[b])
                            hs[b] = [pltpu.async_copy(
                                mref.at[iv.at[b, cc]],
                                buf.at[b, pl.ds(cc * CHUNK, CHUNK)], gs[b])
                                for cc in range(2)]
                    for b in range(NB):
                        it = rb * NB + b

                        @pl.when(it < nk)
                        def _retire(b=b, it=it):
                            for cp in hs[b]:
                                cp.wait()
                            for cc in range(2):
                                pltpu.async_copy(
                                    buf.at[b, pl.ds(cc * CHUNK, CHUNK)],
                                    spm.at[ldv.at[b, cc]], ss[b], add=True)
                    return carry

                lax.fori_loop(0, nro, body, 0)
                for b in range(NB):

                    @pl.when(nk > b)
                    def _final(b=b):
                        pltpu.make_async_copy(mref.at[pl.ds(0, GRP)],
                                              buf.at[b], ss[b]).wait()

                plsc.subcore_barrier()
                pltpu.sync_copy(spm.at[pl.ds(sid * step, step)],
                                out.at[pl.ds(base + sid * step, step)])

            return carry

        lax.fori_loop(0, nround, window, 0)

    return k(m, pexp, ldexp, blo, bhi, zrows)


def _scatter_plan(dest, dr, nwin, nround):
    n = dest.shape[0]
    perm = jnp.argsort(dest).astype(I32)
    s = jnp.take(dest, perm)
    bounds = jnp.searchsorted(s, jnp.arange(nwin + 1) * dr).astype(I32)
    nwin_sz = bounds[1:] - bounds[:-1]
    npad = ((nwin_sz + GRP - 1) // GRP) * GRP
    start_p = jnp.concatenate([jnp.zeros((1,), I32), jnp.cumsum(npad).astype(I32)])
    w_of = s // dr
    pos = jnp.arange(n, dtype=I32) - bounds[w_of] + start_p[w_of]
    ngexp = n // GRP + nwin
    texp = ngexp * GRP
    pexp = jnp.zeros((texp,), I32).at[pos].set(perm).reshape(ngexp, 2, CHUNK)
    ldexp = jnp.full((texp,), dr, I32).at[pos].set(s % dr).reshape(ngexp, 2, CHUNK)
    cb = start_p // GRP

    def pack(a, b):
        v = jnp.stack([a, b])
        return jnp.broadcast_to(v[:, :, None], (2, nround, 16))

    blo = pack(cb[0:nround], cb[nround:nwin])
    bhi = pack(cb[1:nround + 1], cb[nround + 1:nwin + 1])
    return pexp, ldexp, blo, bhi


def kernel(z, rbf, sbf, i, j, idx_kj, idx_ji, params):
    p = params
    i = i.astype(I32)
    j = j.astype(I32)
    idx_kj = idx_kj.astype(I32)
    idx_ji = idx_ji.astype(I32)

    z3 = jnp.pad(z.astype(I32), (0, N_PAD - N_NODES)).reshape(N_GRID, 1, BLK)
    ij2 = jnp.concatenate([i, j]).reshape((2 * N_EDGES) // GRP, 2, CHUNK)
    kj2 = idx_kj.reshape(N_TRI // GRP, 2, CHUNK)
    i2 = i.reshape(N_EDGES // CHUNK, 1, CHUNK)
    peE, ldE, bloE, bhiE = _scatter_plan(idx_ji, DR, NWIN, NROUND)
    zrows = jnp.zeros((N_PAD // 16, H), F32)

    def b1(v):
        return v.reshape(1, H)

    W1 = p['emb_lin_W'][0:H]
    W2 = p['emb_lin_W'][H:2 * H]
    W3 = p['emb_lin_W'][2 * H:3 * H]

    xn = _node_embed(z3, p['emb'])
    xij = _sc_gather(xn, ij2)
    xi = xij[:N_EDGES]
    xj = xij[N_EDGES:]
    x, t = _embed(xi, xj, rbf, p['emb_lin_rbf_W'], b1(p['emb_lin_rbf_b']),
                  W1, W2, W3, b1(p['emb_lin_b']), p['out_lin_rbf'][0])

    def out_block(bb, t_e):
        ns = _sc_scatter_acc(t_e, i2, zrows)
        olp = jnp.pad(p['out_lin'][bb], ((0, 0), (0, H - p['out_lin'][bb].shape[1])))
        return _nmlp(ns, p['out_lins_W'][bb, 0], b1(p['out_lins_b'][bb, 0]),
                     p['out_lins_W'][bb, 1], b1(p['out_lins_b'][bb, 1]),
                     p['out_lins_W'][bb, 2], b1(p['out_lins_b'][bb, 2]), olp)

    P = out_block(0, t)
    for b in range(6):
        xji, xkj = _pre(x, rbf, p['int_lin_ji_W'][b], b1(p['int_lin_ji_b'][b]),
                        p['int_lin_kj_W'][b], b1(p['int_lin_kj_b'][b]),
                        p['int_lin_rbf'][b])
        g = _sc_gather(xkj, kj2)
        U8 = jnp.transpose(p['int_W'][b], (1, 2, 0))
        m = _tri(g, sbf, p['int_lin_sbf'][b], U8)
        agg = _sc_scatter_sorted(m, peE, ldE, bloE, bhiE, zrows, DR, NROUND, N_EDGES)
        weights = [
            p['res_before_W'][b, 0, 0], b1(p['res_before_b'][b, 0, 0]),
            p['res_before_W'][b, 0, 1], b1(p['res_before_b'][b, 0, 1]),
            p['int_lin_W'][b], b1(p['int_lin_b'][b]),
            p['res_after_W'][b, 0, 0], b1(p['res_after_b'][b, 0, 0]),
            p['res_after_W'][b, 0, 1], b1(p['res_after_b'][b, 0, 1]),
            p['res_after_W'][b, 1, 0], b1(p['res_after_b'][b, 1, 0]),
            p['res_after_W'][b, 1, 1], b1(p['res_after_b'][b, 1, 1]),
            p['out_lin_rbf'][b + 1],
        ]
        x, t = _post(agg, xji, x, rbf, weights)
        P = P + out_block(b + 1, t)
    return P[:N_NODES, 0:1]
